# baseline (speedup 1.0000x reference)
import functools

import jax
import jax.numpy as jnp
from jax import lax
from jax.experimental import pallas as pl
from jax.experimental.pallas import tpu as pltpu
from jax.experimental.pallas import tpu_sc as plsc

_VOCAB = 1000000
_EMBED = 64
_EPAD = 128
_BATCH = 16384
_HIST = 50
_B = _BATCH * _HIST

_NC = 2
_NS = 16
_NW = _NC * _NS

_TCOLS = 256
_VMAIN = (_VOCAB // _EPAD) * _EPAD
_NTBLK = _VMAIN // _TCOLS
_TBASE = _NTBLK // _NW
_TEXTRA = _NTBLK - _TBASE * _NW

_mesh = plsc.VectorSubcoreMesh(core_axis_name="c", subcore_axis_name="s")
_params = pltpu.CompilerParams(use_tc_tiling_on_sc=True,
                               needs_layout_passes=False)


@functools.partial(
    pl.kernel,
    mesh=_mesh,
    out_type=jax.ShapeDtypeStruct((_VOCAB, _EPAD), jnp.float32),
    scratch_types=[
        pltpu.VMEM((2, _EMBED, _TCOLS), jnp.float32),
        pltpu.VMEM((2, _TCOLS, _EPAD), jnp.float32),
        pltpu.SemaphoreType.DMA,
        pltpu.SemaphoreType.DMA,
        pltpu.SemaphoreType.DMA,
        pltpu.SemaphoreType.DMA,
    ],
    compiler_params=_params,
)
def _transpose_sc(tt_hbm, tail_hbm, out_hbm, in_v, tr_v,
                  r0, r1, w0, w1):
    rsem = (r0, r1)
    wsem = (w0, w1)
    wid = lax.axis_index("s") * _NC + lax.axis_index("c")
    nblk = _TBASE + jnp.where(wid < _TEXTRA, 1, 0)

    def col0(t):
        return (wid + t * _NW) * _TCOLS

    def issue_read(t, p):
        pltpu.async_copy(tt_hbm.at[:, pl.ds(col0(t), _TCOLS)],
                         in_v.at[p], rsem[p])

    def wait_read(t, p):
        pltpu.make_async_copy(tt_hbm.at[:, pl.ds(col0(t), _TCOLS)],
                              in_v.at[p], rsem[p]).wait()

    def issue_write(t, p):
        pltpu.async_copy(tr_v.at[p], out_hbm.at[pl.ds(col0(t), _TCOLS)],
                         wsem[p])

    def wait_write(t, p):
        pltpu.make_async_copy(tr_v.at[p], out_hbm.at[pl.ds(col0(t), _TCOLS)],
                              wsem[p]).wait()

    def transpose_block(p):
        def body_l(l, carry):
            def body_k(k, c2):
                ee = lax.iota(jnp.int32, 16) + k * 16
                ll = jnp.full((16,), 0, jnp.int32) + l
                vals = plsc.load_gather(in_v.at[p], [ee, ll])
                tr_v[p, l, pl.ds(k * 16, 16)] = vals
                return c2
            return lax.fori_loop(0, _EMBED // 16, body_k, carry)
        lax.fori_loop(0, _TCOLS, body_l, 0)

    issue_read(0, 0)
    issue_read(1, 1)

    for t0 in (0, 1):
        wait_read(t0, t0)
        transpose_block(t0)
        issue_write(t0, t0)
        issue_read(t0 + 2, t0)

    def step(t, carry):
        @pl.when(lax.rem(t, 2) == 0)
        def _():
            wait_read(t, 0)
            wait_write(t - 2, 0)
            transpose_block(0)
            issue_write(t, 0)
            issue_read(t + 2, 0)

        @pl.when(lax.rem(t, 2) == 1)
        def _():
            wait_read(t, 1)
            wait_write(t - 2, 1)
            transpose_block(1)
            issue_write(t, 1)
            issue_read(t + 2, 1)
        return carry

    def tail_step(t, p):
        wait_read(t, p)
        wait_write(t - 2, p)
        transpose_block(p)
        issue_write(t, p)

    lax.fori_loop(2, nblk - 2, step, 0)

    @pl.when(lax.rem(nblk, 2) == 0)
    def _():
        tail_step(nblk - 2, 0)
        tail_step(nblk - 1, 1)
        wait_write(nblk - 2, 0)
        wait_write(nblk - 1, 1)

    @pl.when(lax.rem(nblk, 2) == 1)
    def _():
        tail_step(nblk - 2, 1)
        tail_step(nblk - 1, 0)
        wait_write(nblk - 2, 1)
        wait_write(nblk - 1, 0)

    @pl.when(wid == 0)
    def _():
        pltpu.sync_copy(tail_hbm, in_v.at[0, :, pl.ds(0, _EPAD)])
        pltpu.sync_copy(in_v.at[0, :, pl.ds(0, _EPAD)],
                        out_hbm.at[pl.ds(_VMAIN, _VOCAB - _VMAIN)])


_BROWS_PER_W = _BATCH // _NW
_BBLK = 128
_NJB = _BROWS_PER_W // _BBLK
_NCHUNK = _HIST * _NJB


@functools.partial(
    pl.kernel,
    mesh=_mesh,
    out_type=jax.ShapeDtypeStruct((_HIST, _EMBED, _BATCH), jnp.float32),
    scratch_types=[
        pltpu.VMEM((_BROWS_PER_W * _HIST,), jnp.int32),
        pltpu.VMEM((_HIST, _BROWS_PER_W), jnp.int32),
        pltpu.VMEM((3, _BBLK, _EPAD), jnp.float32),
        pltpu.VMEM((2, _EMBED, _BBLK), jnp.float32),
        pltpu.SemaphoreType.DMA,
        pltpu.SemaphoreType.DMA,
        pltpu.SemaphoreType.DMA,
        pltpu.SemaphoreType.DMA,
        pltpu.SemaphoreType.DMA,
    ],
    compiler_params=_params,
)
def _gather_sc(idx_hbm, table_hbm, out_hbm, idx_v, idxh_v, rows_v, tr_v,
               g0, g1, g2, s0, s1):
    gsem = (g0, g1, g2)
    ssem = (s0, s1)
    wid = lax.axis_index("s") * _NC + lax.axis_index("c")
    span = _BROWS_PER_W * _HIST
    b_base = wid * _BROWS_PER_W

    pltpu.sync_copy(idx_hbm.at[pl.ds(wid * span, span)], idx_v)

    def bucket_h(h, carry):
        def inner(k, c2):
            lanes = (lax.iota(jnp.int32, 16) + k * 16) * _HIST + h
            idxh_v[h, pl.ds(k * 16, 16)] = plsc.load_gather(idx_v, [lanes])
            return c2
        return lax.fori_loop(0, _BROWS_PER_W // 16, inner, carry)
    lax.fori_loop(0, _HIST, bucket_h, 0)

    def chunk_idx(g):
        h = g // _NJB
        jb = lax.rem(g, _NJB)
        return h, jb

    def issue_gather(g, rb):
        h, jb = chunk_idx(g)
        pltpu.async_copy(
            table_hbm.at[idxh_v.at[h, pl.ds(jb * _BBLK, _BBLK)]],
            rows_v.at[rb], gsem[rb])

    def wait_gather(g, rb):
        h, jb = chunk_idx(g)
        pltpu.make_async_copy(
            table_hbm.at[idxh_v.at[h, pl.ds(jb * _BBLK, _BBLK)]],
            rows_v.at[rb], gsem[rb]).wait()

    def issue_store(g, tb):
        h, jb = chunk_idx(g)
        pltpu.async_copy(
            tr_v.at[tb],
            out_hbm.at[h, :, pl.ds(b_base + jb * _BBLK, _BBLK)],
            ssem[tb])

    def wait_store(g, tb):
        h, jb = chunk_idx(g)
        pltpu.make_async_copy(
            tr_v.at[tb],
            out_hbm.at[h, :, pl.ds(b_base + jb * _BBLK, _BBLK)],
            ssem[tb]).wait()

    def transpose_chunk(rb, tb):
        def body_e(e, carry):
            def body_k(k, c2):
                rr = lax.iota(jnp.int32, 16) + k * 16
                cc = jnp.full((16,), 0, jnp.int32) + e
                vals = plsc.load_gather(rows_v.at[rb], [rr, cc])
                tr_v[tb, e, pl.ds(k * 16, 16)] = vals
                return c2
            return lax.fori_loop(0, _BBLK // 16, body_k, carry)
        lax.fori_loop(0, _EMBED, body_e, 0)

    issue_gather(0, 0)
    issue_gather(1, 1)

    for g in (0, 1):
        wait_gather(g, g % 3)
        issue_gather(g + 2, (g + 2) % 3)
        transpose_chunk(g % 3, g % 2)
        issue_store(g, g % 2)

    def six(g6, carry):
        for k in range(6):
            rb = (2 + k) % 3
            tb = k % 2
            g = 2 + g6 * 6 + k
            wait_gather(g, rb)
            issue_gather(g + 2, (2 + k + 2) % 3)
            wait_store(g - 2, tb)
            transpose_chunk(rb, tb)
            issue_store(g, tb)
        return carry

    lax.fori_loop(0, (_NCHUNK - 14) // 6, six, 0)

    for g in range(_NCHUNK - 12, _NCHUNK):
        rb = g % 3
        tb = g % 2
        wait_gather(g, rb)
        if g + 2 < _NCHUNK:
            issue_gather(g + 2, (g + 2) % 3)
        wait_store(g - 2, tb)
        transpose_chunk(rb, tb)
        issue_store(g, tb)
    for g in (_NCHUNK - 2, _NCHUNK - 1):
        wait_store(g, g % 2)


def kernel(x, table):
    flat = x.reshape(-1).astype(jnp.int32)
    tail = jnp.pad(table[_VMAIN:, :], ((0, 0), (0, _EPAD - _EMBED)))
    tpad = _transpose_sc(table.T, tail)
    outT = _gather_sc(flat, tpad)
    return outT.transpose(2, 0, 1)


# speedup vs baseline: 1.1644x; 1.1644x over previous
import functools

import jax
import jax.numpy as jnp
from jax import lax
from jax.experimental import pallas as pl
from jax.experimental.pallas import tpu as pltpu
from jax.experimental.pallas import tpu_sc as plsc

_VOCAB = 1000000
_EMBED = 64
_EPAD = 128
_BATCH = 16384
_HIST = 50
_B = _BATCH * _HIST

_NC = 2
_NS = 16
_NW = _NC * _NS

_TCOLS = 256
_VMAIN = (_VOCAB // _EPAD) * _EPAD
_NTBLK = _VMAIN // _TCOLS
_TBASE = _NTBLK // _NW
_TEXTRA = _NTBLK - _TBASE * _NW

_mesh = plsc.VectorSubcoreMesh(core_axis_name="c", subcore_axis_name="s")
_params = pltpu.CompilerParams(use_tc_tiling_on_sc=True,
                               needs_layout_passes=False)


@functools.partial(
    pl.kernel,
    mesh=_mesh,
    out_type=jax.ShapeDtypeStruct((_VOCAB, _EPAD), jnp.float32),
    scratch_types=[
        pltpu.VMEM((2, _EMBED, _TCOLS), jnp.float32),
        pltpu.VMEM((2, _TCOLS, _EPAD), jnp.float32),
        pltpu.SemaphoreType.DMA,
        pltpu.SemaphoreType.DMA,
        pltpu.SemaphoreType.DMA,
        pltpu.SemaphoreType.DMA,
    ],
    compiler_params=_params,
)
def _transpose_sc(tt_hbm, tail_hbm, out_hbm, in_v, tr_v,
                  r0, r1, w0, w1):
    rsem = (r0, r1)
    wsem = (w0, w1)
    wid = lax.axis_index("s") * _NC + lax.axis_index("c")
    nblk = _TBASE + jnp.where(wid < _TEXTRA, 1, 0)

    def col0(t):
        return (wid + t * _NW) * _TCOLS

    def issue_read(t, p):
        pltpu.async_copy(tt_hbm.at[:, pl.ds(col0(t), _TCOLS)],
                         in_v.at[p], rsem[p])

    def wait_read(t, p):
        pltpu.make_async_copy(tt_hbm.at[:, pl.ds(col0(t), _TCOLS)],
                              in_v.at[p], rsem[p]).wait()

    def issue_write(t, p):
        pltpu.async_copy(tr_v.at[p], out_hbm.at[pl.ds(col0(t), _TCOLS)],
                         wsem[p])

    def wait_write(t, p):
        pltpu.make_async_copy(tr_v.at[p], out_hbm.at[pl.ds(col0(t), _TCOLS)],
                              wsem[p]).wait()

    iota16 = lax.iota(jnp.int32, 16)

    def transpose_block(p):
        def body_e(e, carry):
            ee = jnp.full((16,), 0, jnp.int32) + e
            for k in range(_TCOLS // 16):
                vals = in_v[p, e, pl.ds(k * 16, 16)]
                plsc.store_scatter(tr_v.at[p], [iota16 + k * 16, ee], vals)
            return carry
        lax.fori_loop(0, _EMBED, body_e, 0)

    issue_read(0, 0)
    issue_read(1, 1)

    for t0 in (0, 1):
        wait_read(t0, t0)
        transpose_block(t0)
        issue_write(t0, t0)
        issue_read(t0 + 2, t0)

    def step(t, carry):
        @pl.when(lax.rem(t, 2) == 0)
        def _():
            wait_read(t, 0)
            wait_write(t - 2, 0)
            transpose_block(0)
            issue_write(t, 0)
            issue_read(t + 2, 0)

        @pl.when(lax.rem(t, 2) == 1)
        def _():
            wait_read(t, 1)
            wait_write(t - 2, 1)
            transpose_block(1)
            issue_write(t, 1)
            issue_read(t + 2, 1)
        return carry

    def tail_step(t, p):
        wait_read(t, p)
        wait_write(t - 2, p)
        transpose_block(p)
        issue_write(t, p)

    lax.fori_loop(2, nblk - 2, step, 0)

    @pl.when(lax.rem(nblk, 2) == 0)
    def _():
        tail_step(nblk - 2, 0)
        tail_step(nblk - 1, 1)
        wait_write(nblk - 2, 0)
        wait_write(nblk - 1, 1)

    @pl.when(lax.rem(nblk, 2) == 1)
    def _():
        tail_step(nblk - 2, 1)
        tail_step(nblk - 1, 0)
        wait_write(nblk - 2, 1)
        wait_write(nblk - 1, 0)

    @pl.when(wid == 0)
    def _():
        pltpu.sync_copy(tail_hbm, in_v.at[0, :, pl.ds(0, _EPAD)])
        pltpu.sync_copy(in_v.at[0, :, pl.ds(0, _EPAD)],
                        out_hbm.at[pl.ds(_VMAIN, _VOCAB - _VMAIN)])


_BROWS_PER_W = _BATCH // _NW
_BBLK = 128
_NJB = _BROWS_PER_W // _BBLK
_NCHUNK = _HIST * _NJB


@functools.partial(
    pl.kernel,
    mesh=_mesh,
    out_type=jax.ShapeDtypeStruct((_HIST, _EMBED, _BATCH), jnp.float32),
    scratch_types=[
        pltpu.VMEM((_BROWS_PER_W * _HIST,), jnp.int32),
        pltpu.VMEM((_HIST, _BROWS_PER_W), jnp.int32),
        pltpu.VMEM((3, _BBLK, _EPAD), jnp.float32),
        pltpu.VMEM((2, _EMBED, _BBLK), jnp.float32),
        pltpu.SemaphoreType.DMA,
        pltpu.SemaphoreType.DMA,
        pltpu.SemaphoreType.DMA,
        pltpu.SemaphoreType.DMA,
        pltpu.SemaphoreType.DMA,
    ],
    compiler_params=_params,
)
def _gather_sc(idx_hbm, table_hbm, out_hbm, idx_v, idxh_v, rows_v, tr_v,
               g0, g1, g2, s0, s1):
    gsem = (g0, g1, g2)
    ssem = (s0, s1)
    wid = lax.axis_index("s") * _NC + lax.axis_index("c")
    span = _BROWS_PER_W * _HIST
    b_base = wid * _BROWS_PER_W

    pltpu.sync_copy(idx_hbm.at[pl.ds(wid * span, span)], idx_v)

    iota16 = lax.iota(jnp.int32, 16)

    def bucket_h(h, carry):
        base = iota16 * _HIST + h
        for k in range(_BROWS_PER_W // 16):
            idxh_v[h, pl.ds(k * 16, 16)] = plsc.load_gather(
                idx_v, [base + k * 16 * _HIST])
        return carry
    lax.fori_loop(0, _HIST, bucket_h, 0)

    def chunk_idx(g):
        h = g // _NJB
        jb = lax.rem(g, _NJB)
        return h, jb

    def issue_gather(g, rb):
        h, jb = chunk_idx(g)
        pltpu.async_copy(
            table_hbm.at[idxh_v.at[h, pl.ds(jb * _BBLK, _BBLK)]],
            rows_v.at[rb], gsem[rb])

    def wait_gather(g, rb):
        h, jb = chunk_idx(g)
        pltpu.make_async_copy(
            table_hbm.at[idxh_v.at[h, pl.ds(jb * _BBLK, _BBLK)]],
            rows_v.at[rb], gsem[rb]).wait()

    def issue_store(g, tb):
        h, jb = chunk_idx(g)
        pltpu.async_copy(
            tr_v.at[tb],
            out_hbm.at[h, :, pl.ds(b_base + jb * _BBLK, _BBLK)],
            ssem[tb])

    def wait_store(g, tb):
        h, jb = chunk_idx(g)
        pltpu.make_async_copy(
            tr_v.at[tb],
            out_hbm.at[h, :, pl.ds(b_base + jb * _BBLK, _BBLK)],
            ssem[tb]).wait()

    def transpose_chunk(rb, tb):
        def body_e(e, carry):
            cc = jnp.full((16,), 0, jnp.int32) + e
            for k in range(_BBLK // 16):
                vals = plsc.load_gather(rows_v.at[rb], [iota16 + k * 16, cc])
                tr_v[tb, e, pl.ds(k * 16, 16)] = vals
            return carry
        lax.fori_loop(0, _EMBED, body_e, 0)

    issue_gather(0, 0)
    issue_gather(1, 1)

    for g in (0, 1):
        wait_gather(g, g % 3)
        issue_gather(g + 2, (g + 2) % 3)
        transpose_chunk(g % 3, g % 2)
        issue_store(g, g % 2)

    def six(g6, carry):
        for k in range(6):
            rb = (2 + k) % 3
            tb = k % 2
            g = 2 + g6 * 6 + k
            wait_gather(g, rb)
            issue_gather(g + 2, (2 + k + 2) % 3)
            wait_store(g - 2, tb)
            transpose_chunk(rb, tb)
            issue_store(g, tb)
        return carry

    lax.fori_loop(0, (_NCHUNK - 14) // 6, six, 0)

    for g in range(_NCHUNK - 12, _NCHUNK):
        rb = g % 3
        tb = g % 2
        wait_gather(g, rb)
        if g + 2 < _NCHUNK:
            issue_gather(g + 2, (g + 2) % 3)
        wait_store(g - 2, tb)
        transpose_chunk(rb, tb)
        issue_store(g, tb)
    for g in (_NCHUNK - 2, _NCHUNK - 1):
        wait_store(g, g % 2)


def kernel(x, table):
    flat = x.reshape(-1).astype(jnp.int32)
    tail = jnp.pad(table[_VMAIN:, :], ((0, 0), (0, _EPAD - _EMBED)))
    tpad = _transpose_sc(table.T, tail)
    outT = _gather_sc(flat, tpad)
    return outT.transpose(2, 0, 1)


# speedup vs baseline: 2.2788x; 1.9571x over previous
import functools

import jax
import jax.numpy as jnp
from jax import lax
from jax.experimental import pallas as pl
from jax.experimental.pallas import tpu as pltpu
from jax.experimental.pallas import tpu_sc as plsc

_VOCAB = 1000000
_EMBED = 64
_BATCH = 16384
_HIST = 50
_B = _BATCH * _HIST

_NC = 2
_NS = 16
_NW = _NC * _NS
_B_PER_W = _B // _NW
_CHUNK = 320
_NCHUNK = _B_PER_W // _CHUNK
_NBUF = 4

_mesh = plsc.VectorSubcoreMesh(core_axis_name="c", subcore_axis_name="s")


@functools.partial(
    pl.kernel,
    mesh=_mesh,
    out_type=jax.ShapeDtypeStruct((_B, _EMBED), jnp.float32),
    scratch_types=[
        pltpu.VMEM((_B_PER_W,), jnp.int32),
        pltpu.VMEM((_NBUF, _CHUNK, _EMBED), jnp.float32),
        pltpu.SemaphoreType.DMA,
        pltpu.SemaphoreType.DMA,
        pltpu.SemaphoreType.DMA,
        pltpu.SemaphoreType.DMA,
        pltpu.SemaphoreType.DMA,
        pltpu.SemaphoreType.DMA,
        pltpu.SemaphoreType.DMA,
        pltpu.SemaphoreType.DMA,
    ],
    compiler_params=pltpu.CompilerParams(use_tc_tiling_on_sc=False),
)
def _gather_sc(idx_hbm, table_hbm, out_hbm, idx_v, rows_v,
               g0, g1, g2, g3, s0, s1, s2, s3):
    gsem = (g0, g1, g2, g3)
    ssem = (s0, s1, s2, s3)
    wid = lax.axis_index("s") * _NC + lax.axis_index("c")
    base = wid * _B_PER_W

    pltpu.sync_copy(idx_hbm.at[pl.ds(base, _B_PER_W)], idx_v)

    def idx_slice(g):
        return idx_v.at[pl.ds(g * _CHUNK, _CHUNK)]

    def out_slice(g):
        return out_hbm.at[pl.ds(base + g * _CHUNK, _CHUNK)]

    def issue_gather(g, b):
        pltpu.async_copy(table_hbm.at[idx_slice(g)], rows_v.at[b], gsem[b])

    def wait_gather(g, b):
        pltpu.make_async_copy(table_hbm.at[idx_slice(g)], rows_v.at[b],
                              gsem[b]).wait()

    def issue_store(g, b):
        pltpu.async_copy(rows_v.at[b], out_slice(g), ssem[b])

    def wait_store(g, b):
        pltpu.make_async_copy(rows_v.at[b], out_slice(g), ssem[b]).wait()

    issue_gather(0, 0)
    issue_gather(1, 1)
    wait_gather(0, 0)
    issue_store(0, 0)
    issue_gather(2, 2)
    wait_gather(1, 1)
    issue_store(1, 1)
    issue_gather(3, 3)

    def group(gg, carry):
        for k in range(_NBUF):
            b = (2 + k) % _NBUF
            g = 2 + gg * _NBUF + k
            wait_gather(g, b)
            issue_store(g, b)
            bb = (b + 2) % _NBUF
            wait_store(g - 2, bb)
            issue_gather(g + 2, bb)
        return carry

    lax.fori_loop(0, (_NCHUNK - 4) // _NBUF, group, 0)

    wait_gather(_NCHUNK - 2, (_NCHUNK - 2) % _NBUF)
    issue_store(_NCHUNK - 2, (_NCHUNK - 2) % _NBUF)
    wait_gather(_NCHUNK - 1, (_NCHUNK - 1) % _NBUF)
    issue_store(_NCHUNK - 1, (_NCHUNK - 1) % _NBUF)
    for g in range(_NCHUNK - 4, _NCHUNK):
        wait_store(g, g % _NBUF)


def kernel(x, table):
    flat = x.reshape(-1).astype(jnp.int32)
    out = _gather_sc(flat, table)
    return out.reshape(_BATCH, _HIST, _EMBED)


# speedup vs baseline: 2.4885x; 1.0920x over previous
import functools

import jax
import jax.numpy as jnp
from jax import lax
from jax.experimental import pallas as pl
from jax.experimental.pallas import tpu as pltpu
from jax.experimental.pallas import tpu_sc as plsc

_VOCAB = 1000000
_EMBED = 64
_EPAD = 128
_BATCH = 16384
_HIST = 50
_B = _BATCH * _HIST

_NC = 2
_NS = 16
_NW = _NC * _NS
_BROWS_PER_W = _BATCH // _NW
_CB = 4
_CHUNK = _CB * _HIST
_NCHUNK = _BROWS_PER_W // _CB

_mesh = plsc.VectorSubcoreMesh(core_axis_name="c", subcore_axis_name="s")


@functools.partial(
    pl.kernel,
    mesh=_mesh,
    out_type=jax.ShapeDtypeStruct((_BATCH, _HIST, _EMBED), jnp.float32),
    scratch_types=[
        pltpu.VMEM((3 * _CHUNK,), jnp.int32),
        pltpu.VMEM((3, _CHUNK, _EPAD), jnp.float32),
        pltpu.VMEM((_HIST, _EMBED), jnp.float32),
        pltpu.VMEM((_HIST, _EMBED), jnp.float32),
        pltpu.VMEM((_HIST, _EMBED), jnp.float32),
        pltpu.VMEM((_HIST, _EMBED), jnp.float32),
        pltpu.SemaphoreType.DMA,
        pltpu.SemaphoreType.DMA,
        pltpu.SemaphoreType.DMA,
        pltpu.SemaphoreType.DMA,
        pltpu.SemaphoreType.DMA,
        pltpu.SemaphoreType.DMA,
        pltpu.SemaphoreType.DMA,
    ],
    compiler_params=pltpu.CompilerParams(use_tc_tiling_on_sc=True,
                                         needs_layout_passes=False),
)
def _gather_sc(idx_hbm, table_hbm, out_hbm, idx_v, rows_v,
               t0, t1, t2, t3,
               i0, i1, i2, g0, g1, g2, ssem):
    isem = (i0, i1, i2)
    gsem = (g0, g1, g2)
    t50 = (t0, t1, t2, t3)
    wid = lax.axis_index("s") * _NC + lax.axis_index("c")
    base = wid * _BROWS_PER_W * _HIST
    b_base = wid * _BROWS_PER_W

    def islice(rb):
        return idx_v.at[pl.ds(rb * _CHUNK, _CHUNK)]

    def issue_idx(g, rb):
        pltpu.async_copy(idx_hbm.at[pl.ds(base + g * _CHUNK, _CHUNK)],
                         islice(rb), isem[rb])

    def wait_idx(g, rb):
        pltpu.make_async_copy(idx_hbm.at[pl.ds(base + g * _CHUNK, _CHUNK)],
                              islice(rb), isem[rb]).wait()

    def issue_gather(g, rb):
        pltpu.async_copy(table_hbm.at[islice(rb)], rows_v.at[rb], gsem[rb])

    def wait_gather(g, rb):
        pltpu.make_async_copy(table_hbm.at[islice(rb)], rows_v.at[rb],
                              gsem[rb]).wait()

    def issue_store(g):
        for bb in range(_CB):
            pltpu.async_copy(t50[bb], out_hbm.at[b_base + g * _CB + bb],
                             ssem)

    def wait_store(g):
        for bb in range(_CB):
            pltpu.make_async_copy(t50[bb],
                                  out_hbm.at[b_base + g * _CB + bb],
                                  ssem).wait()

    def repack(rb):
        for bb in range(_CB):
            def body_j(j, carry, bb=bb):
                for k in range(_EMBED // 16):
                    t50[bb][j, pl.ds(k * 16, 16)] = (
                        rows_v[rb, bb * _HIST + j, pl.ds(k * 16, 16)])
                return carry
            lax.fori_loop(0, _HIST, body_j, 0)

    issue_idx(0, 0)
    issue_idx(1, 1)
    issue_idx(2, 2)
    wait_idx(0, 0)
    issue_gather(0, 0)
    wait_idx(1, 1)
    issue_gather(1, 1)

    wait_gather(0, 0)
    issue_idx(3, 0)
    wait_idx(2, 2)
    issue_gather(2, 2)
    repack(0)
    issue_store(0)

    def body(g, rb, nxt):
        wait_gather(g, rb)
        issue_idx(g + 3, rb)
        wait_idx(g + 2, nxt)
        issue_gather(g + 2, nxt)
        wait_store(g - 1)
        repack(rb)
        issue_store(g)

    for g in (1, 2):
        body(g, g % 3, (g + 2) % 3)

    def three(g3, carry):
        for k in range(3):
            g = 3 + g3 * 3 + k
            body(g, k, (k + 2) % 3)
        return carry

    lax.fori_loop(0, (_NCHUNK - 8) // 3, three, 0)

    for g in range(_NCHUNK - 5, _NCHUNK):
        rb = g % 3
        wait_gather(g, rb)
        if g + 3 < _NCHUNK:
            issue_idx(g + 3, rb)
        if g + 2 < _NCHUNK:
            wait_idx(g + 2, (g + 2) % 3)
            issue_gather(g + 2, (g + 2) % 3)
        wait_store(g - 1)
        repack(rb)
        issue_store(g)
    wait_store(_NCHUNK - 1)


def kernel(x, table):
    flat = x.reshape(-1).astype(jnp.int32)
    tpad = jnp.pad(table, ((0, 0), (0, _EPAD - _EMBED)))
    return _gather_sc(flat, tpad)


# speedup vs baseline: 2.4941x; 1.0022x over previous
import functools

import jax
import jax.numpy as jnp
from jax import lax
from jax.experimental import pallas as pl
from jax.experimental.pallas import tpu as pltpu
from jax.experimental.pallas import tpu_sc as plsc

_VOCAB = 1000000
_EMBED = 64
_EPAD = 128
_BATCH = 16384
_HIST = 50
_B = _BATCH * _HIST

_NC = 2
_NS = 16
_NW = _NC * _NS
_BROWS_PER_W = _BATCH // _NW
_CB = 4
_CHUNK = _CB * _HIST
_NCHUNK = _BROWS_PER_W // _CB

_mesh = plsc.VectorSubcoreMesh(core_axis_name="c", subcore_axis_name="s")


@functools.partial(
    pl.kernel,
    mesh=_mesh,
    out_type=jax.ShapeDtypeStruct((_BATCH, _HIST, _EMBED), jnp.float32),
    scratch_types=[
        pltpu.VMEM((3 * _CHUNK,), jnp.int32),
        pltpu.VMEM((3, _CHUNK, _EPAD), jnp.float32),
        pltpu.VMEM((_HIST, _EMBED), jnp.float32),
        pltpu.VMEM((_HIST, _EMBED), jnp.float32),
        pltpu.VMEM((_HIST, _EMBED), jnp.float32),
        pltpu.VMEM((_HIST, _EMBED), jnp.float32),
        pltpu.SemaphoreType.DMA,
        pltpu.SemaphoreType.DMA,
        pltpu.SemaphoreType.DMA,
        pltpu.SemaphoreType.DMA,
        pltpu.SemaphoreType.DMA,
        pltpu.SemaphoreType.DMA,
        pltpu.SemaphoreType.DMA,
    ],
    compiler_params=pltpu.CompilerParams(use_tc_tiling_on_sc=True,
                                         needs_layout_passes=False),
)
def _gather_sc(idx_hbm, table_hbm, out_hbm, idx_v, rows_v,
               t0, t1, t2, t3,
               i0, i1, i2, g0, g1, g2, ssem):
    isem = (i0, i1, i2)
    gsem = (g0, g1, g2)
    t50 = (t0, t1, t2, t3)
    wid = lax.axis_index("s") * _NC + lax.axis_index("c")
    base = wid * _BROWS_PER_W * _HIST
    b_base = wid * _BROWS_PER_W

    def islice(rb):
        return idx_v.at[pl.ds(rb * _CHUNK, _CHUNK)]

    def issue_idx(g, rb):
        pltpu.async_copy(idx_hbm.at[pl.ds(base + g * _CHUNK, _CHUNK)],
                         islice(rb), isem[rb])

    def wait_idx(g, rb):
        pltpu.make_async_copy(idx_hbm.at[pl.ds(base + g * _CHUNK, _CHUNK)],
                              islice(rb), isem[rb]).wait()

    def issue_gather(g, rb):
        pltpu.async_copy(table_hbm.at[islice(rb)], rows_v.at[rb], gsem[rb])

    def wait_gather(g, rb):
        pltpu.make_async_copy(table_hbm.at[islice(rb)], rows_v.at[rb],
                              gsem[rb]).wait()

    def issue_store(g):
        for bb in range(_CB):
            pltpu.async_copy(t50[bb], out_hbm.at[b_base + g * _CB + bb],
                             ssem)

    def wait_store(g):
        for bb in range(_CB):
            pltpu.make_async_copy(t50[bb],
                                  out_hbm.at[b_base + g * _CB + bb],
                                  ssem).wait()

    def repack(rb):
        def body_j(j, carry):
            for bb in range(_CB):
                for k in range(_EMBED // 16):
                    t50[bb][j, pl.ds(k * 16, 16)] = (
                        rows_v[rb, bb * _HIST + j, pl.ds(k * 16, 16)])
            return carry
        lax.fori_loop(0, _HIST, body_j, 0)

    issue_idx(0, 0)
    issue_idx(1, 1)
    issue_idx(2, 2)
    wait_idx(0, 0)
    issue_gather(0, 0)
    wait_idx(1, 1)
    issue_gather(1, 1)

    wait_gather(0, 0)
    issue_idx(3, 0)
    wait_idx(2, 2)
    issue_gather(2, 2)
    repack(0)
    issue_store(0)

    def body(g, rb, nxt):
        wait_gather(g, rb)
        issue_idx(g + 3, rb)
        wait_idx(g + 2, nxt)
        issue_gather(g + 2, nxt)
        wait_store(g - 1)
        repack(rb)
        issue_store(g)

    for g in (1, 2):
        body(g, g % 3, (g + 2) % 3)

    def three(g3, carry):
        for k in range(3):
            g = 3 + g3 * 3 + k
            body(g, k, (k + 2) % 3)
        return carry

    lax.fori_loop(0, (_NCHUNK - 8) // 3, three, 0)

    for g in range(_NCHUNK - 5, _NCHUNK):
        rb = g % 3
        wait_gather(g, rb)
        if g + 3 < _NCHUNK:
            issue_idx(g + 3, rb)
        if g + 2 < _NCHUNK:
            wait_idx(g + 2, (g + 2) % 3)
            issue_gather(g + 2, (g + 2) % 3)
        wait_store(g - 1)
        repack(rb)
        issue_store(g)
    wait_store(_NCHUNK - 1)


def kernel(x, table):
    flat = x.reshape(-1).astype(jnp.int32)
    tpad = jnp.pad(table, ((0, 0), (0, _EPAD - _EMBED)))
    return _gather_sc(flat, tpad)
